# baseline (speedup 1.0000x reference)
import functools

import jax
import jax.numpy as jnp
from jax import lax
from jax.experimental import pallas as pl
from jax.experimental.pallas import tpu as pltpu
from jax.experimental.pallas import tpu_sc as plsc

_HIDDEN = 4096
_EXPERTS = 64
_K = 8
_BLOCK = 1024

_NC = 2
_NS = 16
_NW = _NC * _NS
_GRP = 16


def _softmax_block(x_ref, w_ref, out_ref):
    logits = jax.lax.dot_general(
        w_ref[...], x_ref[...], (((1,), (1,)), ((), ())),
        preferred_element_type=jnp.float32)
    m = jnp.max(logits, axis=0, keepdims=True)
    e = jnp.exp(logits - m)
    out_ref[...] = e / jnp.sum(e, axis=0, keepdims=True)


def _tc_softmax(x, W):
    tokens = x.shape[0]
    return pl.pallas_call(
        _softmax_block,
        grid=(tokens // _BLOCK,),
        in_specs=[
            pl.BlockSpec((_BLOCK, _HIDDEN), lambda i: (i, 0)),
            pl.BlockSpec((_EXPERTS, _HIDDEN), lambda i: (0, 0)),
        ],
        out_specs=pl.BlockSpec((_EXPERTS, _BLOCK), lambda i: (0, i)),
        out_shape=jax.ShapeDtypeStruct((_EXPERTS, tokens), jnp.float32),
    )(x, W)


def _sc_topk(wts):
    tokens = wts.shape[1]
    per_w = tokens // _NW
    ngroups = per_w // _GRP
    mesh = plsc.VectorSubcoreMesh(core_axis_name="c", subcore_axis_name="s")

    tile = 128
    ntiles = per_w // tile
    nsub = tile // _GRP

    @functools.partial(
        pl.kernel, mesh=mesh,
        out_type=[jax.ShapeDtypeStruct((_K, tokens), jnp.float32),
                  jax.ShapeDtypeStruct((_K, tokens), jnp.int32)],
        scratch_types=[pltpu.VMEM((_EXPERTS, tile), jnp.float32),
                       pltpu.VMEM((_K, tile), jnp.float32),
                       pltpu.VMEM((_K, tile), jnp.int32)])
    def sc_kernel(w_hbm, tw_hbm, te_hbm, wbuf, twbuf, tebuf):
        wid = lax.axis_index("s") * _NC + lax.axis_index("c")
        base = wid * per_w

        @pl.loop(0, ntiles)
        def _(g):
            col = base + g * tile
            pltpu.sync_copy(w_hbm.at[:, pl.ds(col, tile)], wbuf)
            for sg in range(nsub):
                sl = pl.ds(sg * _GRP, _GRP)
                keys = []
                for e2 in range(_EXPERTS):
                    bits = lax.bitcast_convert_type(wbuf[e2, sl], jnp.int32)
                    keys.append(jnp.bitwise_or(
                        jnp.bitwise_and(bits, jnp.int32(-64)),
                        jnp.int32(_EXPERTS - 1 - e2)))
                prev = jnp.full((_GRP,), jnp.iinfo(jnp.int32).max, jnp.int32)
                sel_v = []
                sel_e = []
                for _k in range(_K):
                    best = jnp.full((_GRP,), -1, jnp.int32)
                    for e2 in range(_EXPERTS):
                        best = jnp.maximum(
                            best,
                            jnp.where(keys[e2] < prev, keys[e2],
                                      jnp.int32(-1)))
                    prev = best
                    sel_e.append(_EXPERTS - 1 -
                                 jnp.bitwise_and(best, jnp.int32(63)))
                    sel_v.append(lax.bitcast_convert_type(
                        jnp.bitwise_and(best, jnp.int32(-64)), jnp.float32))
                total = sel_v[0]
                for _k in range(1, _K):
                    total = total + sel_v[_k]
                for _k in range(_K):
                    twbuf[_k, sl] = sel_v[_k] / total
                    tebuf[_k, sl] = sel_e[_k]
            pltpu.sync_copy(twbuf, tw_hbm.at[:, pl.ds(col, tile)])
            pltpu.sync_copy(tebuf, te_hbm.at[:, pl.ds(col, tile)])

    return sc_kernel(wts)


def kernel(x, W):
    wts = _tc_softmax(x, W)
    twt, tet = _sc_topk(wts)
    return twt.T, tet.T


# speedup vs baseline: 1.5357x; 1.5357x over previous
import jax
import jax.numpy as jnp
from jax.experimental import pallas as pl
from jax.experimental.pallas import tpu as pltpu

_HIDDEN = 4096
_EXPERTS = 64
_K = 8
_BLOCK = 1024


def _router_block(x_ref, w_ref, tw_ref, te_ref):
    x = x_ref[...]
    w = w_ref[...]
    logits = jax.lax.dot_general(
        w, x, (((1,), (1,)), ((), ())), preferred_element_type=jnp.float32)
    eidx = jax.lax.broadcasted_iota(jnp.int32, logits.shape, 0)
    l = logits
    vals = []
    idxs = []
    for _ in range(_K):
        m = jnp.max(l, axis=0, keepdims=True)
        idx = jnp.min(jnp.where(l == m, eidx, _EXPERTS),
                      axis=0, keepdims=True)
        vals.append(m)
        idxs.append(idx)
        l = jnp.where(eidx == idx, -jnp.inf, l)
    v = jnp.concatenate(vals, axis=0)
    e = jnp.exp(v - v[0:1])
    wts = e / jnp.sum(e, axis=0, keepdims=True)
    tw_ref[...] = wts.T
    te_ref[...] = jnp.concatenate(idxs, axis=0).T


def kernel(x, W):
    tokens = x.shape[0]
    grid = (tokens // _BLOCK,)
    tw, te = pl.pallas_call(
        _router_block,
        grid=grid,
        in_specs=[
            pl.BlockSpec((_BLOCK, _HIDDEN), lambda i: (i, 0)),
            pl.BlockSpec((_EXPERTS, _HIDDEN), lambda i: (0, 0)),
        ],
        out_specs=[
            pl.BlockSpec((_BLOCK, _K), lambda i: (i, 0)),
            pl.BlockSpec((_BLOCK, _K), lambda i: (i, 0)),
        ],
        out_shape=[
            jax.ShapeDtypeStruct((tokens, _K), jnp.float32),
            jax.ShapeDtypeStruct((tokens, _K), jnp.int32),
        ],
    )(x, W)
    return tw, te
